# baseline (speedup 1.0000x reference)
import jax
import jax.numpy as jnp
from jax.experimental import pallas as pl
from jax.experimental.pallas import tpu as pltpu

_TID = 288
_DIW = 7
_BB = 8


def _st_kernel(xt_ref, idx_ref, w1t_ref, wtt_ref, wdt_ref, bt_ref, o_ref):
    n = xt_ref.shape[2]
    row_t = jax.lax.broadcasted_iota(jnp.int32, (_TID, n), 0)
    row_d = jax.lax.broadcasted_iota(jnp.int32, (8, n), 0)
    bias = bt_ref[...]
    for j in range(_BB):
        f = xt_ref[j].astype(jnp.bfloat16)
        tid = idx_ref[j, 0]
        diw = idx_ref[j, 1]
        oh_t = (row_t == tid[None, :]).astype(jnp.bfloat16)
        oh_d = (row_d == diw[None, :]).astype(jnp.bfloat16)
        acc = jnp.dot(w1t_ref[...], f, preferred_element_type=jnp.float32)
        acc = acc + jnp.dot(wtt_ref[...], oh_t, preferred_element_type=jnp.float32)
        acc = acc + jnp.dot(wdt_ref[...], oh_d, preferred_element_type=jnp.float32)
        o_ref[j] = acc + bias


def kernel(x, w_conv, w_tab, bias_node):
    B, L, N, C = x.shape
    K, Eo = w_conv.shape

    xt = x.reshape(B, K, N)
    tid = jnp.clip((x[:, -1, :, 1] * 288.0).astype(jnp.int32), 0, _TID - 1)
    diw = jnp.clip(x[:, -1, :, 2].astype(jnp.int32), 0, _DIW - 1)
    idx = jnp.stack([tid, diw], axis=1)

    w1t = w_conv.T.astype(jnp.bfloat16)
    wtt = w_tab[:_TID].T.astype(jnp.bfloat16)
    wdt = w_tab[_TID:_TID + 8].T.astype(jnp.bfloat16)
    biast = bias_node.T

    out = pl.pallas_call(
        _st_kernel,
        out_shape=jax.ShapeDtypeStruct((B, Eo, N), jnp.float32),
        grid=(B // _BB,),
        in_specs=[
            pl.BlockSpec((_BB, K, N), lambda i: (i, 0, 0)),
            pl.BlockSpec((_BB, 2, N), lambda i: (i, 0, 0)),
            pl.BlockSpec((Eo, K), lambda i: (0, 0)),
            pl.BlockSpec((Eo, _TID), lambda i: (0, 0)),
            pl.BlockSpec((Eo, 8), lambda i: (0, 0)),
            pl.BlockSpec((Eo, N), lambda i: (0, 0)),
        ],
        out_specs=pl.BlockSpec((_BB, Eo, N), lambda i: (i, 0, 0)),
        compiler_params=pltpu.CompilerParams(dimension_semantics=("parallel",)),
    )(xt, idx, w1t, wtt, wdt, biast)

    return out[..., None]


# speedup vs baseline: 1.3557x; 1.3557x over previous
import jax
import jax.numpy as jnp
from jax.experimental import pallas as pl
from jax.experimental.pallas import tpu as pltpu

_TID = 288
_DIW = 7
_BB = 16


def _st_kernel(xt_ref, idx_ref, w1t_ref, wtt_ref, wdt_ref, bt_ref, o_ref):
    n = xt_ref.shape[2]
    row_t = jax.lax.broadcasted_iota(jnp.int32, (_TID, n), 0)
    row_d = jax.lax.broadcasted_iota(jnp.int32, (8, n), 0)
    bias = bt_ref[...]
    for j in range(_BB):
        f = xt_ref[j]
        tid = idx_ref[j, 0]
        diw = idx_ref[j, 1]
        oh_t = (row_t == tid[None, :]).astype(jnp.bfloat16)
        oh_d = (row_d == diw[None, :]).astype(jnp.bfloat16)
        acc = jnp.dot(w1t_ref[...], f, preferred_element_type=jnp.float32)
        acc = acc + jnp.dot(wtt_ref[...], oh_t, preferred_element_type=jnp.float32)
        acc = acc + jnp.dot(wdt_ref[...], oh_d, preferred_element_type=jnp.float32)
        o_ref[j] = acc + bias


def kernel(x, w_conv, w_tab, bias_node):
    B, L, N, C = x.shape
    K, Eo = w_conv.shape

    xt = jnp.transpose(x, (0, 1, 3, 2)).reshape(B, K, N).astype(jnp.bfloat16)
    tid = jnp.clip((x[:, -1, :, 1] * 288.0).astype(jnp.int32), 0, _TID - 1)
    diw = jnp.clip(x[:, -1, :, 2].astype(jnp.int32), 0, _DIW - 1)
    idx = jnp.stack([tid, diw], axis=1)

    w1t = w_conv.T.astype(jnp.bfloat16)
    wtt = w_tab[:_TID].T.astype(jnp.bfloat16)
    wdt = w_tab[_TID:_TID + 8].T.astype(jnp.bfloat16)
    biast = bias_node.T

    out = pl.pallas_call(
        _st_kernel,
        out_shape=jax.ShapeDtypeStruct((B, Eo, N), jnp.float32),
        grid=(B // _BB,),
        in_specs=[
            pl.BlockSpec((_BB, K, N), lambda i: (i, 0, 0)),
            pl.BlockSpec((_BB, 2, N), lambda i: (i, 0, 0)),
            pl.BlockSpec((Eo, K), lambda i: (0, 0)),
            pl.BlockSpec((Eo, _TID), lambda i: (0, 0)),
            pl.BlockSpec((Eo, 8), lambda i: (0, 0)),
            pl.BlockSpec((Eo, N), lambda i: (0, 0)),
        ],
        out_specs=pl.BlockSpec((_BB, Eo, N), lambda i: (i, 0, 0)),
        compiler_params=pltpu.CompilerParams(dimension_semantics=("parallel",)),
    )(xt, idx, w1t, wtt, wdt, biast)

    return out[..., None]


# speedup vs baseline: 1.3560x; 1.0002x over previous
import jax
import jax.numpy as jnp
from jax.experimental import pallas as pl
from jax.experimental.pallas import tpu as pltpu

_TID = 288
_DIW = 7
_BB = 32


def _st_kernel(xt_ref, idx_ref, w1t_ref, wtt_ref, wdt_ref, bt_ref, o_ref):
    n = xt_ref.shape[2]
    row_t = jax.lax.broadcasted_iota(jnp.int32, (_TID, n), 0)
    row_d = jax.lax.broadcasted_iota(jnp.int32, (8, n), 0)
    bias = bt_ref[...]
    for j in range(_BB):
        f = xt_ref[j]
        tid = idx_ref[j, 0]
        diw = idx_ref[j, 1]
        oh_t = (row_t == tid[None, :]).astype(jnp.bfloat16)
        oh_d = (row_d == diw[None, :]).astype(jnp.bfloat16)
        acc = jnp.dot(w1t_ref[...], f, preferred_element_type=jnp.float32)
        acc = acc + jnp.dot(wtt_ref[...], oh_t, preferred_element_type=jnp.float32)
        acc = acc + jnp.dot(wdt_ref[...], oh_d, preferred_element_type=jnp.float32)
        o_ref[j] = acc + bias


def kernel(x, w_conv, w_tab, bias_node):
    B, L, N, C = x.shape
    K, Eo = w_conv.shape

    xt = jnp.transpose(x, (0, 1, 3, 2)).reshape(B, K, N).astype(jnp.bfloat16)
    tid = jnp.clip((x[:, -1, :, 1] * 288.0).astype(jnp.int32), 0, _TID - 1)
    diw = jnp.clip(x[:, -1, :, 2].astype(jnp.int32), 0, _DIW - 1)
    idx = jnp.stack([tid, diw], axis=1)

    w1t = w_conv.T.astype(jnp.bfloat16)
    wtt = w_tab[:_TID].T.astype(jnp.bfloat16)
    wdt = w_tab[_TID:_TID + 8].T.astype(jnp.bfloat16)
    biast = bias_node.T

    out = pl.pallas_call(
        _st_kernel,
        out_shape=jax.ShapeDtypeStruct((B, Eo, N), jnp.float32),
        grid=(B // _BB,),
        in_specs=[
            pl.BlockSpec((_BB, K, N), lambda i: (i, 0, 0)),
            pl.BlockSpec((_BB, 2, N), lambda i: (i, 0, 0)),
            pl.BlockSpec((Eo, K), lambda i: (0, 0)),
            pl.BlockSpec((Eo, _TID), lambda i: (0, 0)),
            pl.BlockSpec((Eo, 8), lambda i: (0, 0)),
            pl.BlockSpec((Eo, N), lambda i: (0, 0)),
        ],
        out_specs=pl.BlockSpec((_BB, Eo, N), lambda i: (i, 0, 0)),
        compiler_params=pltpu.CompilerParams(dimension_semantics=("parallel",)),
    )(xt, idx, w1t, wtt, wdt, biast)

    return out[..., None]


# speedup vs baseline: 1.8381x; 1.3555x over previous
import jax
import jax.numpy as jnp
from jax.experimental import pallas as pl
from jax.experimental.pallas import tpu as pltpu

_TID = 288
_DIW = 7
_BB = 4


def _st_kernel(xt_ref, idx_ref, w1t_ref, wtt_ref, wdt_ref, bt_ref, o_ref):
    n = xt_ref.shape[2]
    row_t = jax.lax.broadcasted_iota(jnp.int32, (_TID, n), 0)
    row_d = jax.lax.broadcasted_iota(jnp.int32, (8, n), 0)
    bias = bt_ref[...]
    for j in range(_BB):
        f = xt_ref[j]
        tid = idx_ref[j, 0]
        diw = idx_ref[j, 1]
        oh_t = (row_t == tid[None, :]).astype(jnp.bfloat16)
        oh_d = (row_d == diw[None, :]).astype(jnp.bfloat16)
        acc = jnp.dot(w1t_ref[...], f, preferred_element_type=jnp.float32)
        acc = acc + jnp.dot(wtt_ref[...], oh_t, preferred_element_type=jnp.float32)
        acc = acc + jnp.dot(wdt_ref[...], oh_d, preferred_element_type=jnp.float32)
        o_ref[j, :, 0, :] = acc + bias


def kernel(x, w_conv, w_tab, bias_node):
    B, L, N, C = x.shape
    K, Eo = w_conv.shape

    xt = jnp.transpose(x, (0, 1, 3, 2)).reshape(B, K, N).astype(jnp.bfloat16)
    tid = jnp.clip((x[:, -1, :, 1] * 288.0).astype(jnp.int32), 0, _TID - 1)
    diw = jnp.clip(x[:, -1, :, 2].astype(jnp.int32), 0, _DIW - 1)
    idx = jnp.stack([tid, diw], axis=1)

    w1t = w_conv.T.astype(jnp.bfloat16)
    wtt = w_tab[:_TID].T.astype(jnp.bfloat16)
    wdt = w_tab[_TID:_TID + 8].T.astype(jnp.bfloat16)
    biast = bias_node.T

    out = pl.pallas_call(
        _st_kernel,
        out_shape=jax.ShapeDtypeStruct((B, Eo, 1, N), jnp.float32),
        grid=(B // _BB,),
        in_specs=[
            pl.BlockSpec((_BB, K, N), lambda i: (i, 0, 0)),
            pl.BlockSpec((_BB, 2, N), lambda i: (i, 0, 0)),
            pl.BlockSpec((Eo, K), lambda i: (0, 0)),
            pl.BlockSpec((Eo, _TID), lambda i: (0, 0)),
            pl.BlockSpec((Eo, 8), lambda i: (0, 0)),
            pl.BlockSpec((Eo, N), lambda i: (0, 0)),
        ],
        out_specs=pl.BlockSpec((_BB, Eo, 1, N), lambda i: (i, 0, 0, 0)),
        compiler_params=pltpu.CompilerParams(dimension_semantics=("parallel",)),
    )(xt, idx, w1t, wtt, wdt, biast)

    return jnp.transpose(out, (0, 1, 3, 2))


# speedup vs baseline: 2.1643x; 1.1775x over previous
import jax
import jax.numpy as jnp
from jax.experimental import pallas as pl
from jax.experimental.pallas import tpu as pltpu

_TID = 288
_DIW = 7
_BB = 8


def _st_kernel(xt_ref, idx_ref, w1t_ref, wtt_ref, wdt_ref, bt_ref, o_ref):
    n = xt_ref.shape[2]
    row_t = jax.lax.broadcasted_iota(jnp.int32, (_TID, n), 0)
    row_d = jax.lax.broadcasted_iota(jnp.int32, (8, n), 0)
    bias = bt_ref[...]
    for j in range(_BB):
        f = xt_ref[j]
        tid = idx_ref[j, 0]
        diw = idx_ref[j, 1]
        oh_t = (row_t == tid[None, :]).astype(jnp.bfloat16)
        oh_d = (row_d == diw[None, :]).astype(jnp.bfloat16)
        acc = jnp.dot(w1t_ref[...], f, preferred_element_type=jnp.float32)
        acc = acc + jnp.dot(wtt_ref[...], oh_t, preferred_element_type=jnp.float32)
        acc = acc + jnp.dot(wdt_ref[...], oh_d, preferred_element_type=jnp.float32)
        o_ref[j, :, 0, :] = acc + bias


def kernel(x, w_conv, w_tab, bias_node):
    B, L, N, C = x.shape
    K, Eo = w_conv.shape

    xt = jnp.transpose(x, (0, 1, 3, 2)).reshape(B, K, N).astype(jnp.bfloat16)
    tid = jnp.clip((x[:, -1, :, 1] * 288.0).astype(jnp.int32), 0, _TID - 1)
    diw = jnp.clip(x[:, -1, :, 2].astype(jnp.int32), 0, _DIW - 1)
    idx = jnp.stack([tid, diw], axis=1)

    w1t = w_conv.T.astype(jnp.bfloat16)
    wtt = w_tab[:_TID].T.astype(jnp.bfloat16)
    wdt = w_tab[_TID:_TID + 8].T.astype(jnp.bfloat16)
    biast = bias_node.T

    out = pl.pallas_call(
        _st_kernel,
        out_shape=jax.ShapeDtypeStruct((B, Eo, 1, N), jnp.float32),
        grid=(B // _BB,),
        in_specs=[
            pl.BlockSpec((_BB, K, N), lambda i: (i, 0, 0)),
            pl.BlockSpec((_BB, 2, N), lambda i: (i, 0, 0)),
            pl.BlockSpec((Eo, K), lambda i: (0, 0)),
            pl.BlockSpec((Eo, _TID), lambda i: (0, 0)),
            pl.BlockSpec((Eo, 8), lambda i: (0, 0)),
            pl.BlockSpec((Eo, N), lambda i: (0, 0)),
        ],
        out_specs=pl.BlockSpec((_BB, Eo, 1, N), lambda i: (i, 0, 0, 0)),
        compiler_params=pltpu.CompilerParams(dimension_semantics=("parallel",)),
    )(xt, idx, w1t, wtt, wdt, biast)

    return jnp.transpose(out, (0, 1, 3, 2))
